# baseline (speedup 1.0000x reference)
import numpy as np
import jax
import jax.numpy as jnp
from jax.experimental import pallas as pl
from jax.experimental.pallas import tpu as pltpu

_K = 64
_ROWS = 48
_W = 384
_P = _ROWS * _W


def _perm_matrices():
    m = np.zeros((384, 384), np.float32)
    for c in range(3):
        for l in range(128):
            m[128 * c + l, 3 * l + c] = 1.0
    return jnp.asarray(m), jnp.asarray(m.T.copy())


def _tc_body(a_ref, c_ref, m_ref, de_ref, x_ref, o_ref, scr_ref, scr2_ref):
    for s in range(3):
        d_s = x_ref[0][:, 384 * s:384 * s + 384]
        planar_s = jax.lax.dot_general(
            d_s, de_ref[...], (((1,), (0,)), ((), ())),
            preferred_element_type=jnp.float32)
        for c in range(3):
            scr2_ref[c, :, 128 * s:128 * s + 128] = \
                planar_s[:, 128 * c:128 * c + 128]
    xt = jnp.stack(
        [jnp.concatenate([scr2_ref[c, h, :] for h in range(_ROWS)], axis=0)
         for c in range(3)])

    logits = jax.lax.dot_general(
        a_ref[0], xt, (((1,), (0,)), ((), ())),
        preferred_element_type=jnp.float32)
    e = jnp.exp2(logits)
    num = jax.lax.dot_general(
        c_ref[0], e, (((1,), (0,)), ((), ())),
        preferred_element_type=jnp.float32)
    inv = 1.0 / num[3]
    q = num[0:3] * inv[None, :]

    for c in range(3):
        for h in range(_ROWS):
            scr_ref[c, h, :] = q[c, _W * h:_W * h + _W]
    runs = []
    for s in range(3):
        pos = jnp.concatenate(
            [scr_ref[c][:, 128 * s:128 * s + 128] for c in range(3)],
            axis=1)
        runs.append(jax.lax.dot_general(
            pos, m_ref[...], (((1,), (0,)), ((), ())),
            preferred_element_type=jnp.float32))
    o_ref[0] = jnp.concatenate(runs, axis=1)


def kernel(images, palettes, temperature):
    B, H, W, C = images.shape
    n = H * W

    x2 = images.reshape(B, H, W * C)

    inv_t = 1.0 / temperature
    ln2 = 0.6931471805599453
    a = (2.0 * inv_t / ln2) * palettes
    wb = jnp.exp(-inv_t * jnp.sum(palettes * palettes, -1))
    cp = jnp.concatenate(
        [(palettes * wb[:, :, None]).transpose(0, 2, 1),
         wb[:, None, :]], axis=1)

    mi, mde = _perm_matrices()
    grid = (B, H // _ROWS)
    out = pl.pallas_call(
        _tc_body,
        grid=grid,
        in_specs=[
            pl.BlockSpec((1, _K, C), lambda bi, i: (bi, 0, 0)),
            pl.BlockSpec((1, 4, _K), lambda bi, i: (bi, 0, 0)),
            pl.BlockSpec((384, 384), lambda bi, i: (0, 0)),
            pl.BlockSpec((384, 384), lambda bi, i: (0, 0)),
            pl.BlockSpec((1, _ROWS, W * C), lambda bi, i: (bi, i, 0)),
        ],
        out_specs=pl.BlockSpec((1, _ROWS, W * C), lambda bi, i: (bi, i, 0)),
        out_shape=jax.ShapeDtypeStruct((B, H, W * C), jnp.float32),
        scratch_shapes=[pltpu.VMEM((3, _ROWS, _W), jnp.float32),
                        pltpu.VMEM((3, _ROWS, _W), jnp.float32)],
    )(a, cp, mi, mde, x2)

    return out.reshape(B, H, W, C)


# speedup vs baseline: 1.2138x; 1.2138x over previous
import numpy as np
import jax
import jax.numpy as jnp
from jax.experimental import pallas as pl
from jax.experimental.pallas import tpu as pltpu

_K = 64
_ROWS = 96
_W = 384
_P = _ROWS * _W


def _interleave_matrix():
    m = np.zeros((384, 384), np.float32)
    for c in range(3):
        for l in range(128):
            m[128 * c + l, 3 * l + c] = 1.0
    return jnp.asarray(m)


def _tc_body(a_ref, c_ref, m_ref, x_ref, o_ref, scr_ref):
    logits = jax.lax.dot_general(
        a_ref[0], x_ref[0], (((1,), (0,)), ((), ())),
        preferred_element_type=jnp.float32)
    e = jnp.exp2(logits)
    num = jax.lax.dot_general(
        c_ref[0], e, (((1,), (0,)), ((), ())),
        preferred_element_type=jnp.float32)
    inv = 1.0 / num[3]
    q = num[0:3] * inv[None, :]
    for c in range(3):
        for h in range(_ROWS):
            scr_ref[c, h, :] = q[c, _W * h:_W * h + _W]
    grids = [scr_ref[c] for c in range(3)]
    runs = []
    for s in range(3):
        pos = jnp.concatenate(
            [g[:, 128 * s:128 * s + 128] for g in grids], axis=1)
        runs.append(jax.lax.dot_general(
            pos, m_ref[...], (((1,), (0,)), ((), ())),
            preferred_element_type=jnp.float32))
    o_ref[0] = jnp.concatenate(runs, axis=1)


def kernel(images, palettes, temperature):
    B, H, W, C = images.shape
    n = H * W

    x = images.reshape(B, n, C).transpose(0, 2, 1)

    inv_t = 1.0 / temperature
    ln2 = 0.6931471805599453
    a = (2.0 * inv_t / ln2) * palettes
    wb = jnp.exp(-inv_t * jnp.sum(palettes * palettes, -1))
    cp = jnp.concatenate(
        [(palettes * wb[:, :, None]).transpose(0, 2, 1),
         wb[:, None, :]], axis=1)

    grid = (B, n // _P)
    out = pl.pallas_call(
        _tc_body,
        grid=grid,
        in_specs=[
            pl.BlockSpec((1, _K, C), lambda bi, i: (bi, 0, 0)),
            pl.BlockSpec((1, 4, _K), lambda bi, i: (bi, 0, 0)),
            pl.BlockSpec((384, 384), lambda bi, i: (0, 0)),
            pl.BlockSpec((1, C, _P), lambda bi, i: (bi, 0, i)),
        ],
        out_specs=pl.BlockSpec((1, _ROWS, W * C), lambda bi, i: (bi, i, 0)),
        out_shape=jax.ShapeDtypeStruct((B, H, W * C), jnp.float32),
        scratch_shapes=[pltpu.VMEM((3, _ROWS, _W), jnp.float32)],
    )(a, cp, _interleave_matrix(), x)

    return out.reshape(B, H, W, C)


# speedup vs baseline: 1.2527x; 1.0320x over previous
import numpy as np
import jax
import jax.numpy as jnp
from jax.experimental import pallas as pl
from jax.experimental.pallas import tpu as pltpu

_K = 64
_ROWS = 192
_W = 384
_P = _ROWS * _W


def _interleave_matrix():
    m = np.zeros((384, 384), np.float32)
    for c in range(3):
        for l in range(128):
            m[128 * c + l, 3 * l + c] = 1.0
    return jnp.asarray(m)


def _tc_body(a_ref, c_ref, m_ref, x_ref, o_ref, scr_ref):
    logits = jax.lax.dot_general(
        a_ref[0], x_ref[0], (((1,), (0,)), ((), ())),
        preferred_element_type=jnp.float32)
    e = jnp.exp2(logits)
    num = jax.lax.dot_general(
        c_ref[0], e, (((1,), (0,)), ((), ())),
        preferred_element_type=jnp.float32)
    inv = 1.0 / num[3]
    q = num[0:3] * inv[None, :]
    for c in range(3):
        for h in range(_ROWS):
            scr_ref[c, h, :] = q[c, _W * h:_W * h + _W]
    grids = [scr_ref[c] for c in range(3)]
    runs = []
    for s in range(3):
        pos = jnp.concatenate(
            [g[:, 128 * s:128 * s + 128] for g in grids], axis=1)
        runs.append(jax.lax.dot_general(
            pos, m_ref[...], (((1,), (0,)), ((), ())),
            preferred_element_type=jnp.float32))
    o_ref[0] = jnp.concatenate(runs, axis=1)


def kernel(images, palettes, temperature):
    B, H, W, C = images.shape
    n = H * W

    x = images.reshape(B, n, C).transpose(0, 2, 1)

    inv_t = 1.0 / temperature
    ln2 = 0.6931471805599453
    a = (2.0 * inv_t / ln2) * palettes
    wb = jnp.exp(-inv_t * jnp.sum(palettes * palettes, -1))
    cp = jnp.concatenate(
        [(palettes * wb[:, :, None]).transpose(0, 2, 1),
         wb[:, None, :]], axis=1)

    grid = (B, n // _P)
    out = pl.pallas_call(
        _tc_body,
        grid=grid,
        in_specs=[
            pl.BlockSpec((1, _K, C), lambda bi, i: (bi, 0, 0)),
            pl.BlockSpec((1, 4, _K), lambda bi, i: (bi, 0, 0)),
            pl.BlockSpec((384, 384), lambda bi, i: (0, 0)),
            pl.BlockSpec((1, C, _P), lambda bi, i: (bi, 0, i)),
        ],
        out_specs=pl.BlockSpec((1, _ROWS, W * C), lambda bi, i: (bi, i, 0)),
        out_shape=jax.ShapeDtypeStruct((B, H, W * C), jnp.float32),
        scratch_shapes=[pltpu.VMEM((3, _ROWS, _W), jnp.float32)],
    )(a, cp, _interleave_matrix(), x)

    return out.reshape(B, H, W, C)


# speedup vs baseline: 1.2673x; 1.0117x over previous
import numpy as np
import jax
import jax.numpy as jnp
from jax.experimental import pallas as pl
from jax.experimental.pallas import tpu as pltpu

_K = 64
_ROWS = 384
_W = 384
_P = _ROWS * _W


def _interleave_matrix():
    m = np.zeros((384, 384), np.float32)
    for c in range(3):
        for l in range(128):
            m[128 * c + l, 3 * l + c] = 1.0
    return jnp.asarray(m)


def _tc_body(a_ref, c_ref, m_ref, x_ref, o_ref, scr_ref):
    logits = jax.lax.dot_general(
        a_ref[0], x_ref[0], (((1,), (0,)), ((), ())),
        preferred_element_type=jnp.float32)
    e = jnp.exp2(logits)
    num = jax.lax.dot_general(
        c_ref[0], e, (((1,), (0,)), ((), ())),
        preferred_element_type=jnp.float32)
    inv = 1.0 / num[3]
    q = num[0:3] * inv[None, :]
    for c in range(3):
        for h in range(_ROWS):
            scr_ref[c, h, :] = q[c, _W * h:_W * h + _W]
    grids = [scr_ref[c] for c in range(3)]
    runs = []
    for s in range(3):
        pos = jnp.concatenate(
            [g[:, 128 * s:128 * s + 128] for g in grids], axis=1)
        runs.append(jax.lax.dot_general(
            pos, m_ref[...], (((1,), (0,)), ((), ())),
            preferred_element_type=jnp.float32))
    o_ref[0] = jnp.concatenate(runs, axis=1)


def kernel(images, palettes, temperature):
    B, H, W, C = images.shape
    n = H * W

    x = images.reshape(B, n, C).transpose(0, 2, 1)

    inv_t = 1.0 / temperature
    ln2 = 0.6931471805599453
    a = (2.0 * inv_t / ln2) * palettes
    wb = jnp.exp(-inv_t * jnp.sum(palettes * palettes, -1))
    cp = jnp.concatenate(
        [(palettes * wb[:, :, None]).transpose(0, 2, 1),
         wb[:, None, :]], axis=1)

    grid = (B, n // _P)
    out = pl.pallas_call(
        _tc_body,
        grid=grid,
        in_specs=[
            pl.BlockSpec((1, _K, C), lambda bi, i: (bi, 0, 0)),
            pl.BlockSpec((1, 4, _K), lambda bi, i: (bi, 0, 0)),
            pl.BlockSpec((384, 384), lambda bi, i: (0, 0)),
            pl.BlockSpec((1, C, _P), lambda bi, i: (bi, 0, i)),
        ],
        out_specs=pl.BlockSpec((1, _ROWS, W * C), lambda bi, i: (bi, i, 0)),
        out_shape=jax.ShapeDtypeStruct((B, H, W * C), jnp.float32),
        scratch_shapes=[pltpu.VMEM((3, _ROWS, _W), jnp.float32)],
    )(a, cp, _interleave_matrix(), x)

    return out.reshape(B, H, W, C)
